# Initial kernel scaffold; baseline (speedup 1.0000x reference)
import jax
import jax.numpy as jnp
from jax.experimental import pallas as pl


def kernel(feature_maps, W, b):
    raise NotImplementedError("write your pallas kernel here")


# speedup vs baseline: 1.6496x; 1.6496x over previous
import functools

import jax
import jax.numpy as jnp
from jax import lax
from jax.experimental import pallas as pl
from jax.experimental.pallas import tpu as pltpu
from jax.experimental.pallas import tpu_sc as plsc

B, C, H, W_DIM = 16, 96, 224, 224
HW = H * W_DIM
M = int(HW * 0.25)
NCHUNK = 8
S = HW // NCHUNK
NVEC = HW // 16

_f32 = jnp.float32
_i32 = jnp.int32


def _att_body(fm_ref, w_ref, b_ref, att_ref):
    x = fm_ref[0]
    w = w_ref[...]
    acc = lax.dot_general(w, x, (((1,), (0,)), ((), ())),
                          preferred_element_type=_f32)
    att_ref[0] = jax.nn.sigmoid(acc + b_ref[0, 0])


def _compute_att(fm3, w2, b2):
    return pl.pallas_call(
        _att_body,
        grid=(B, NCHUNK),
        in_specs=[
            pl.BlockSpec((1, C, S), lambda i, j: (i, 0, j)),
            pl.BlockSpec((1, C), lambda i, j: (0, 0)),
            pl.BlockSpec(memory_space=pltpu.SMEM),
        ],
        out_specs=pl.BlockSpec((1, 1, S), lambda i, j: (i, 0, j)),
        out_shape=jax.ShapeDtypeStruct((B, 1, HW), _f32),
    )(fm3, w2, b2)


def _bcast_i32(x, n=16):
    return lax.broadcast(jnp.asarray(x, _i32), (n,))


def _bcast_f32(x, n=16):
    return lax.broadcast(jnp.asarray(x, _f32), (n,))


def _sc_body(att_hbm, out_hbm, data_v, hist_v, tvec_v):
    wid = lax.axis_index("s") * 2 + lax.axis_index("c")

    @pl.when(wid < B)
    def _():
        pltpu.sync_copy(att_hbm.at[wid], data_v)

        lanes = lax.iota(_i32, 16)
        ones = jnp.ones((16,), _f32)
        zeros = jnp.zeros((16,), _f32)
        m255 = _bcast_i32(255)

        def histogram(shift, prefix, check):
            def zero_body(i, _):
                hist_v[pl.ds(i * 16, 16)] = zeros
                return 0
            lax.fori_loop(0, 16, zero_body, 0)
            sh_v = _bcast_i32(shift)
            shp_v = _bcast_i32(shift + 8)
            pv = lax.broadcast(prefix, (16,))

            def body(j, _):
                v = data_v[pl.ds(j * 16, 16)]
                bits = lax.bitcast_convert_type(v, _i32)
                bn = lax.shift_right_logical(bits, sh_v) & m255
                if check:
                    msk = lax.shift_right_logical(bits, shp_v) == pv
                    plsc.addupdate_scatter(hist_v, [bn], ones, mask=msk)
                else:
                    plsc.addupdate_scatter(hist_v, [bn], ones)
                return 0
            lax.fori_loop(0, NVEC, body, 0)

        def pick_bin(rem):
            def bs(i, acc):
                hv = hist_v[pl.ds(i * 16, 16)]
                s = jnp.sum(hv)
                iv = lax.broadcast(i, (16,))
                return acc + jnp.where(lanes == iv, lax.broadcast(s, (16,)), zeros)
            bsum = lax.fori_loop(0, 16, bs, zeros)
            gsb = jnp.flip(jnp.cumsum(jnp.flip(bsum)))
            remv = lax.broadcast(rem, (16,))
            istar = (jnp.sum(jnp.where(gsb >= remv, ones, zeros)) - 1.0).astype(_i32)
            iv = lax.broadcast(istar, (16,))
            after = (jnp.sum(jnp.where(lanes == iv, gsb, zeros))
                     - jnp.sum(jnp.where(lanes == iv, bsum, zeros)))

            def pk(i, acc):
                hv = hist_v[pl.ds(i * 16, 16)]
                sel = lax.broadcast(i, (16,)) == iv
                return jnp.where(sel, hv, acc)
            selv = lax.fori_loop(0, 16, pk, zeros)

            wgs = jnp.flip(jnp.cumsum(jnp.flip(selv))) + lax.broadcast(after, (16,))
            jstar = (jnp.sum(jnp.where(wgs >= remv, ones, zeros)) - 1.0).astype(_i32)
            jv = lax.broadcast(jstar, (16,))
            sel_wgs = jnp.sum(jnp.where(lanes == jv, wgs, zeros))
            sel_h = jnp.sum(jnp.where(lanes == jv, selv, zeros))
            rem_new = rem - (sel_wgs - sel_h)
            return istar * 16 + jstar, rem_new

        rem = jnp.asarray(float(M), _f32)
        prefix = jnp.asarray(0, _i32)
        for lvl, shift in enumerate((24, 16, 8, 0)):
            histogram(shift, prefix, check=(lvl > 0))
            binstar, rem = pick_bin(rem)
            prefix = prefix * 256 + binstar

        tvec_v[...] = lax.bitcast_convert_type(lax.broadcast(prefix, (16,)), _f32)
        pltpu.sync_copy(tvec_v, out_hbm.at[wid])


_sc_select = functools.partial(
    pl.kernel,
    out_type=jax.ShapeDtypeStruct((B, 16), _f32),
    mesh=plsc.VectorSubcoreMesh(core_axis_name="c", subcore_axis_name="s",
                                num_cores=2, num_subcores=16),
    compiler_params=pltpu.CompilerParams(needs_layout_passes=False),
    scratch_types=[
        pltpu.VMEM((HW,), _f32),
        pltpu.VMEM((256,), _f32),
        pltpu.VMEM((16,), _f32),
    ],
)(_sc_body)


def _mask_body(fm_ref, att_ref, thr_ref, out_ref):
    t = thr_ref[pl.program_id(0), 0]
    keep = (att_ref[0] < t).astype(_f32)
    out_ref[0] = fm_ref[0] * keep


def _apply_mask(fm3, att3, thr):
    return pl.pallas_call(
        _mask_body,
        grid=(B, NCHUNK),
        in_specs=[
            pl.BlockSpec((1, C, S), lambda i, j: (i, 0, j)),
            pl.BlockSpec((1, 1, S), lambda i, j: (i, 0, j)),
            pl.BlockSpec(memory_space=pltpu.SMEM),
        ],
        out_specs=pl.BlockSpec((1, C, S), lambda i, j: (i, 0, j)),
        out_shape=jax.ShapeDtypeStruct((B, C, HW), _f32),
    )(fm3, att3, thr)


def kernel(feature_maps, W, b):
    fm3 = feature_maps.reshape(B, C, HW)
    w2 = W.reshape(1, C)
    b2 = b.reshape(1, 1)
    att3 = _compute_att(fm3, w2, b2)
    thr = _sc_select(att3.reshape(B, HW))
    out = _apply_mask(fm3, att3, thr)
    return (out.reshape(B, C, H, W_DIM),
            att3.reshape(B, 1, H, W_DIM))
